# baseline (speedup 1.0000x reference)
import functools

import jax
import jax.numpy as jnp
from jax import lax
from jax.experimental import pallas as pl
from jax.experimental.pallas import tpu as pltpu
from jax.experimental.pallas import tpu_sc as plsc

_L = 16


def _diffusion_call(Tm1, E, N):
    mesh = plsc.VectorSubcoreMesh(
        core_axis_name="c", subcore_axis_name="s", num_cores=2, num_subcores=16)
    NS = 16
    KE = Tm1 * E
    KCH = KE // NS
    E3 = 3 * E

    @functools.partial(
        pl.kernel,
        out_type=(
            jax.ShapeDtypeStruct((Tm1 + 1, N), jnp.float32),
            jax.ShapeDtypeStruct((KE,), jnp.float32),
            jax.ShapeDtypeStruct((KE,), jnp.float32),
        ),
        mesh=mesh,
        compiler_params=pltpu.CompilerParams(needs_layout_passes=False),
        scratch_types=[
            pltpu.VMEM((N,), jnp.float32),
            pltpu.VMEM((2 * E3,), jnp.int32),
            pltpu.VMEM((E,), jnp.float32),
            pltpu.VMEM((3 * _L,), jnp.float32),
            pltpu.SemaphoreType.DMA,
            pltpu.SemaphoreType.DMA,
            pltpu.SemaphoreType.DMA,
        ],
    )
    def body(e_hbm, lx_hbm, prm_hbm,
             x_out, kap_out, d_out,
             x_v, e_v, dmu_v, prm_v, sem_x, sem_e, sem_d):
        cid = lax.axis_index("c")
        sid = lax.axis_index("s")
        lanes = lax.iota(jnp.int32, _L)
        p3 = lanes * 3

        @pl.when(jnp.logical_and(cid == 0, sid == 0))
        def _():
            pltpu.sync_copy(prm_hbm, prm_v)
            mu = prm_v[pl.ds(_L, _L)]
            pltpu.sync_copy(lx_hbm, x_v)

            @plsc.parallel_loop(0, N, _L, unroll=4)
            def _(i):
                z = x_v[pl.ds(i, _L)]
                x_v[pl.ds(i, _L)] = 1.0 / (1.0 + jnp.exp(-z))

            pltpu.async_copy(x_v, x_out.at[0], sem_x)
            pltpu.sync_copy(e_hbm.at[pl.ds(0, E3)], e_v.at[pl.ds(0, E3)])
            pltpu.async_copy(dmu_v, d_out.at[pl.ds(0, E)], sem_d)

            def step(t, c):
                par = lax.rem(t, 2)
                npar = 1 - par
                base = par * E3

                t1 = jnp.minimum(t + 1, Tm1 - 1)
                pltpu.async_copy(e_hbm.at[pl.ds(t1 * E3, E3)],
                                 e_v.at[pl.ds(npar * E3, E3)], sem_e)

                pltpu.make_async_copy(
                    dmu_v, d_out.at[pl.ds(0, E)], sem_d).wait()

                @plsc.parallel_loop(0, E, _L, unroll=4)
                def _(e):
                    pu = p3 + (base + 3 * e)
                    iu = plsc.load_gather(e_v, [pu])
                    iv = plsc.load_gather(e_v, [pu + 1])
                    dd = plsc.load_gather(x_v, [iu]) - plsc.load_gather(x_v, [iv])
                    dmu_v[pl.ds(e, _L)] = dd * mu

                pltpu.async_copy(dmu_v, d_out.at[pl.ds(t * E, E)], sem_d)

                pltpu.make_async_copy(x_v, x_out.at[0], sem_x).wait()

                @plsc.parallel_loop(0, E, _L, unroll=4)
                def _(e):
                    pu = p3 + (base + 3 * e)
                    iu = plsc.load_gather(e_v, [pu])
                    iv = plsc.load_gather(e_v, [pu + 1])
                    dm = dmu_v[pl.ds(e, _L)]
                    plsc.addupdate_scatter(x_v, [iv], dm)
                    plsc.addupdate_scatter(x_v, [iu], -dm)

                pltpu.async_copy(x_v, x_out.at[t + 1], sem_x)

                pltpu.make_async_copy(e_hbm.at[pl.ds(0, E3)],
                                      e_v.at[pl.ds(0, E3)], sem_e).wait()
                return c

            lax.fori_loop(0, Tm1, step, 0)
            pltpu.make_async_copy(dmu_v, d_out.at[pl.ds(0, E)], sem_d).wait()
            pltpu.make_async_copy(x_v, x_out.at[0], sem_x).wait()

        @pl.when(cid == 0)
        def _():
            pltpu.sync_copy(prm_hbm, prm_v)
            th = prm_v[pl.ds(0, _L)]
            mu = prm_v[pl.ds(_L, _L)]
            rho = prm_v[pl.ds(2 * _L, _L)]
            eps = 1.0 / (1.0 + jnp.exp(-th))
            bias = rho * eps
            scale = rho / mu
            plsc.subcore_barrier()
            base = sid * KCH
            pltpu.sync_copy(d_out.at[pl.ds(base, KCH)], x_v.at[pl.ds(0, KCH)])

            @plsc.parallel_loop(0, KCH, _L, unroll=4)
            def _(i):
                dm = x_v[pl.ds(i, _L)]
                z = bias - scale * jnp.abs(dm)
                x_v[pl.ds(i, _L)] = 1.0 / (1.0 + jnp.exp(-z))

            pltpu.sync_copy(x_v.at[pl.ds(0, KCH)], kap_out.at[pl.ds(base, KCH)])

    return body


def kernel(edges, logit_X0, theta, mu, rho):
    Tm1, E, _ = edges.shape
    N = logit_X0.shape[0]
    e3 = edges.reshape(-1)
    prm = jnp.concatenate([
        jnp.broadcast_to(theta.astype(jnp.float32), (_L,)),
        jnp.full((_L,), mu, jnp.float32),
        jnp.full((_L,), rho, jnp.float32),
    ])
    X, kap, _ = _diffusion_call(Tm1, E, N)(
        e3, logit_X0.astype(jnp.float32), prm)
    return X, kap


# speedup vs baseline: 1.9369x; 1.9369x over previous
import functools

import jax
import jax.numpy as jnp
from jax import lax
from jax.experimental import pallas as pl
from jax.experimental.pallas import tpu as pltpu
from jax.experimental.pallas import tpu_sc as plsc

_L = 16


def _diffusion_call(Tm1, E, N):
    mesh = plsc.VectorSubcoreMesh(
        core_axis_name="c", subcore_axis_name="s", num_cores=2, num_subcores=16)
    NS = 16
    KE = Tm1 * E
    KCH = KE // NS

    @functools.partial(
        pl.kernel,
        out_type=(
            jax.ShapeDtypeStruct((Tm1 + 1, N), jnp.float32),
            jax.ShapeDtypeStruct((KE,), jnp.float32),
            jax.ShapeDtypeStruct((KE,), jnp.float32),
        ),
        mesh=mesh,
        compiler_params=pltpu.CompilerParams(needs_layout_passes=False),
        scratch_types=[
            pltpu.VMEM((N,), jnp.float32),
            pltpu.VMEM((2, 2 * E), jnp.int32),
            pltpu.VMEM((E,), jnp.float32),
            pltpu.VMEM((3 * _L,), jnp.float32),
            pltpu.SemaphoreType.DMA,
            pltpu.SemaphoreType.DMA,
            pltpu.SemaphoreType.DMA,
        ],
    )
    def body(uv_hbm, lx_hbm, prm_hbm,
             x_out, kap_out, d_out,
             x_v, uv_v, dmu_v, prm_v, sem_x, sem_e, sem_d):
        cid = lax.axis_index("c")
        sid = lax.axis_index("s")

        @pl.when(jnp.logical_and(cid == 0, sid == 0))
        def _():
            pltpu.sync_copy(prm_hbm, prm_v)
            mu = prm_v[pl.ds(_L, _L)]
            pltpu.sync_copy(lx_hbm, x_v)

            @plsc.parallel_loop(0, N, _L, unroll=4)
            def _(i):
                z = x_v[pl.ds(i, _L)]
                x_v[pl.ds(i, _L)] = 1.0 / (1.0 + jnp.exp(-z))

            pltpu.async_copy(x_v, x_out.at[0], sem_x)
            pltpu.sync_copy(uv_hbm.at[0], uv_v.at[0])
            pltpu.async_copy(dmu_v, d_out.at[pl.ds(0, E)], sem_d)

            def step(t, c):
                par = lax.rem(t, 2)
                npar = 1 - par

                t1 = jnp.minimum(t + 1, Tm1 - 1)
                pltpu.async_copy(uv_hbm.at[t1], uv_v.at[npar], sem_e)

                pltpu.make_async_copy(
                    dmu_v, d_out.at[pl.ds(0, E)], sem_d).wait()

                @plsc.parallel_loop(0, E, _L, unroll=4)
                def _(e):
                    iu = uv_v[par, pl.ds(e, _L)]
                    iv = uv_v[par, pl.ds(E + e, _L)]
                    dd = plsc.load_gather(x_v, [iu]) - plsc.load_gather(x_v, [iv])
                    dmu_v[pl.ds(e, _L)] = dd * mu

                pltpu.async_copy(dmu_v, d_out.at[pl.ds(t * E, E)], sem_d)

                pltpu.make_async_copy(x_v, x_out.at[0], sem_x).wait()

                @plsc.parallel_loop(0, E, _L, unroll=4)
                def _(e):
                    iu = uv_v[par, pl.ds(e, _L)]
                    iv = uv_v[par, pl.ds(E + e, _L)]
                    dm = dmu_v[pl.ds(e, _L)]
                    plsc.addupdate_scatter(x_v, [iv], dm)
                    plsc.addupdate_scatter(x_v, [iu], -dm)

                pltpu.async_copy(x_v, x_out.at[t + 1], sem_x)

                pltpu.make_async_copy(uv_hbm.at[0], uv_v.at[0], sem_e).wait()
                return c

            lax.fori_loop(0, Tm1, step, 0)
            pltpu.make_async_copy(dmu_v, d_out.at[pl.ds(0, E)], sem_d).wait()
            pltpu.make_async_copy(x_v, x_out.at[0], sem_x).wait()

        @pl.when(cid == 0)
        def _():
            pltpu.sync_copy(prm_hbm, prm_v)
            th = prm_v[pl.ds(0, _L)]
            mu = prm_v[pl.ds(_L, _L)]
            rho = prm_v[pl.ds(2 * _L, _L)]
            eps = 1.0 / (1.0 + jnp.exp(-th))
            bias = rho * eps
            scale = rho / mu
            plsc.subcore_barrier()
            base = sid * KCH
            pltpu.sync_copy(d_out.at[pl.ds(base, KCH)], x_v.at[pl.ds(0, KCH)])

            @plsc.parallel_loop(0, KCH, _L, unroll=4)
            def _(i):
                dm = x_v[pl.ds(i, _L)]
                z = bias - scale * jnp.abs(dm)
                x_v[pl.ds(i, _L)] = 1.0 / (1.0 + jnp.exp(-z))

            pltpu.sync_copy(x_v.at[pl.ds(0, KCH)], kap_out.at[pl.ds(base, KCH)])

    return body


def kernel(edges, logit_X0, theta, mu, rho):
    Tm1, E, _ = edges.shape
    N = logit_X0.shape[0]
    uv = jnp.concatenate([edges[:, :, 0], edges[:, :, 1]], axis=1)
    prm = jnp.concatenate([
        jnp.broadcast_to(theta.astype(jnp.float32), (_L,)),
        jnp.full((_L,), mu, jnp.float32),
        jnp.full((_L,), rho, jnp.float32),
    ])
    X, kap, _ = _diffusion_call(Tm1, E, N)(
        uv, logit_X0.astype(jnp.float32), prm)
    return X, kap
